# Initial kernel scaffold; baseline (speedup 1.0000x reference)
import jax
import jax.numpy as jnp
from jax.experimental import pallas as pl


def kernel(hidden_states, router_w, gate_up_proj, down_proj, shared_gate_w, shared_up_w, shared_down_w):
    raise NotImplementedError("write your pallas kernel here")


# speedup vs baseline: 1.6285x; 1.6285x over previous
import functools

import jax
import jax.numpy as jnp
from jax import lax
from jax.experimental import pallas as pl
from jax.experimental.pallas import tpu as pltpu
from jax.experimental.pallas import tpu_sc as plsc

HIDDEN = 1024
FF = 2048
E = 8
T = 2048
BT = 128
NA = 2 * T
NB = NA // BT + E
NR = NB * BT
NC, NS = 2, 16
NW = NC * NS
CHUNK = 32


def _route_body(x_ref, rw_ref, xs_ref, pos_ref, be_ref):
    x = x_ref[...]
    rw = rw_ref[...]
    logits = lax.dot_general(
        x.astype(jnp.bfloat16), rw.astype(jnp.bfloat16),
        (((1,), (1,)), ((), ())),
        preferred_element_type=jnp.float32)
    col = lax.broadcasted_iota(jnp.int32, (T, E), 1)
    m1 = jnp.max(logits, axis=1, keepdims=True)
    idx1 = jnp.min(jnp.where(logits == m1, col, E), axis=1, keepdims=True)
    sel1 = col == idx1
    neg = jnp.where(sel1, -jnp.inf, logits)
    m2 = jnp.max(neg, axis=1, keepdims=True)
    idx2 = jnp.min(jnp.where(neg == m2, col, E), axis=1, keepdims=True)
    sel2 = col == idx2
    xs_ref[0:T, :] = x * jax.nn.sigmoid(m1)
    xs_ref[T:2 * T, :] = x * jax.nn.sigmoid(m2)

    msel = (sel1 | sel2).astype(jnp.float32)
    C = 256
    carry = jnp.zeros((1, E), jnp.float32)
    ranks = []
    li = lax.broadcasted_iota(jnp.int32, (C, C), 0)
    lj = lax.broadcasted_iota(jnp.int32, (C, C), 1)
    ltri = (lj < li).astype(jnp.float32)
    for c in range(T // C):
        blk = msel[c * C:(c + 1) * C, :]
        r = lax.dot_general(ltri, blk, (((1,), (0,)), ((), ())),
                            preferred_element_type=jnp.float32) + carry
        ranks.append(r)
        carry = carry + jnp.sum(blk, axis=0, keepdims=True)
    rank = jnp.concatenate(ranks, axis=0)
    counts = carry
    padded = jnp.ceil(counts * (1.0 / BT)) * BT
    ei = lax.broadcasted_iota(jnp.int32, (E, E), 0)
    ej = lax.broadcasted_iota(jnp.int32, (E, E), 1)
    etri = (ei < ej).astype(jnp.float32)
    start = lax.dot_general(padded, etri, (((1,), (0,)), ((), ())),
                            preferred_element_type=jnp.float32)
    posf = jnp.broadcast_to(start, (T, E)) + rank
    pos1 = jnp.sum(jnp.where(sel1, posf, 0.0), axis=1)
    pos2 = jnp.sum(jnp.where(sel2, posf, 0.0), axis=1)
    pos_ref[0, :] = pos1.astype(jnp.int32)
    pos_ref[1, :] = pos2.astype(jnp.int32)
    bi = (lax.broadcasted_iota(jnp.int32, (NB, E), 0) * BT).astype(jnp.float32)
    be = jnp.sum((jnp.broadcast_to(start, (NB, E)) <= bi).astype(jnp.int32),
                 axis=1) - 1
    be_ref[0, :] = be


_route = pl.pallas_call(
    _route_body,
    out_shape=[
        jax.ShapeDtypeStruct((NA, HIDDEN), jnp.float32),
        jax.ShapeDtypeStruct((2, T), jnp.int32),
        jax.ShapeDtypeStruct((1, NB), jnp.int32),
    ],
)


def _scatter_rows_body(xsc_hbm, pos_hbm, out_hbm, xv, pv, sem):
    wid = lax.axis_index("s") * NC + lax.axis_index("c")
    per_w = NA // NW
    for sub in range(per_w // CHUNK):
        base = wid * per_w + sub * CHUNK
        pltpu.sync_copy(pos_hbm.at[pl.ds(base, CHUNK)], pv)
        pltpu.sync_copy(xsc_hbm.at[pl.ds(base, CHUNK)], xv)
        pltpu.async_copy(xv, out_hbm.at[pv], sem).wait()


@functools.cache
def _sc_kernels():
    mesh = plsc.VectorSubcoreMesh(core_axis_name="c", subcore_axis_name="s")
    scatter_rows = pl.kernel(
        _scatter_rows_body, mesh=mesh,
        out_type=jax.ShapeDtypeStruct((NR, HIDDEN), jnp.float32),
        scratch_types=[
            pltpu.VMEM((CHUNK, HIDDEN), jnp.float32),
            pltpu.VMEM((CHUNK,), jnp.int32),
            pltpu.SemaphoreType.DMA,
        ],
    )
    combine = pl.kernel(
        _combine_body, mesh=mesh,
        out_type=jax.ShapeDtypeStruct((T, HIDDEN), jnp.float32),
        scratch_types=[
            pltpu.VMEM((CHUNK, HIDDEN), jnp.float32),
            pltpu.VMEM((CHUNK, HIDDEN), jnp.float32),
            pltpu.VMEM((CHUNK, HIDDEN), jnp.float32),
            pltpu.VMEM((CHUNK,), jnp.int32),
            pltpu.VMEM((CHUNK,), jnp.int32),
            pltpu.SemaphoreType.DMA,
        ],
    )
    return scatter_rows, combine


def _experts_body(be_ref, xs_ref, gu_ref, dn_ref, y_ref):
    xb = xs_ref[...]
    gu = lax.dot_general(xb, gu_ref[0], (((1,), (0,)), ((), ())),
                         preferred_element_type=jnp.float32)
    gate = gu[:, :FF]
    up = gu[:, FF:]
    h = up * (gate * jax.nn.sigmoid(gate))
    y_ref[...] = lax.dot_general(h, dn_ref[0], (((1,), (0,)), ((), ())),
                                 preferred_element_type=jnp.float32)


_experts = pl.pallas_call(
    _experts_body,
    grid_spec=pltpu.PrefetchScalarGridSpec(
        num_scalar_prefetch=1,
        grid=(NB,),
        in_specs=[
            pl.BlockSpec((BT, HIDDEN), lambda b, be: (b, 0)),
            pl.BlockSpec((1, HIDDEN, 2 * FF), lambda b, be: (be[b], 0, 0)),
            pl.BlockSpec((1, FF, HIDDEN), lambda b, be: (be[b], 0, 0)),
        ],
        out_specs=pl.BlockSpec((BT, HIDDEN), lambda b, be: (b, 0)),
    ),
    out_shape=jax.ShapeDtypeStruct((NR, HIDDEN), jnp.float32),
)


def _shared_body(x_ref, g_ref, u_ref, d_ref, o_ref):
    xb = x_ref[...]
    g = lax.dot_general(xb, g_ref[...], (((1,), (1,)), ((), ())),
                        preferred_element_type=jnp.float32)
    u = lax.dot_general(xb, u_ref[...], (((1,), (1,)), ((), ())),
                        preferred_element_type=jnp.float32)
    h = (g * jax.nn.sigmoid(g)) * u
    o_ref[...] = lax.dot_general(h, d_ref[...], (((1,), (1,)), ((), ())),
                                 preferred_element_type=jnp.float32)


_BTS = 512
_shared = pl.pallas_call(
    _shared_body,
    grid=(T // _BTS,),
    in_specs=[
        pl.BlockSpec((_BTS, HIDDEN), lambda i: (i, 0)),
        pl.BlockSpec((FF, HIDDEN), lambda i: (0, 0)),
        pl.BlockSpec((FF, HIDDEN), lambda i: (0, 0)),
        pl.BlockSpec((HIDDEN, FF), lambda i: (0, 0)),
    ],
    out_specs=pl.BlockSpec((_BTS, HIDDEN), lambda i: (i, 0)),
    out_shape=jax.ShapeDtypeStruct((T, HIDDEN), jnp.float32),
)


def _combine_body(y_hbm, p1_hbm, p2_hbm, sh_hbm, out_hbm,
                  shv, y1v, y2v, p1v, p2v, sem):
    wid = lax.axis_index("s") * NC + lax.axis_index("c")
    per_w = T // NW
    for sub in range(per_w // CHUNK):
        base = wid * per_w + sub * CHUNK
        pltpu.sync_copy(p1_hbm.at[pl.ds(base, CHUNK)], p1v)
        pltpu.sync_copy(p2_hbm.at[pl.ds(base, CHUNK)], p2v)
        pltpu.sync_copy(sh_hbm.at[pl.ds(base, CHUNK)], shv)
        cp1 = pltpu.async_copy(y_hbm.at[p1v], y1v, sem)
        cp2 = pltpu.async_copy(y_hbm.at[p2v], y2v, sem)
        cp1.wait()
        cp2.wait()

        def row(i, _):
            def vec(d, _):
                sl = pl.ds(d * 16, 16)
                shv[i, sl] = shv[i, sl] + y1v[i, sl] + y2v[i, sl]
                return 0
            return lax.fori_loop(0, HIDDEN // 16, vec, 0)

        lax.fori_loop(0, CHUNK, row, 0)
        pltpu.sync_copy(shv, out_hbm.at[pl.ds(base, CHUNK)])


def kernel(hidden_states, router_w, gate_up_proj, down_proj,
           shared_gate_w, shared_up_w, shared_down_w):
    x = hidden_states.reshape(T, HIDDEN)
    scatter_rows, combine = _sc_kernels()
    xsc, posarr, be = _route(x, router_w)
    xs = scatter_rows(xsc, posarr.reshape(NA))
    y = _experts(be.reshape(NB), xs, gate_up_proj, down_proj)
    sh = _shared(x, shared_gate_w, shared_up_w, shared_down_w)
    out = combine(y, posarr[0], posarr[1], sh)
    return out.reshape(1, T, HIDDEN)


# speedup vs baseline: 1.7838x; 1.0954x over previous
import functools

import jax
import jax.numpy as jnp
from jax import lax
from jax.experimental import pallas as pl
from jax.experimental.pallas import tpu as pltpu
from jax.experimental.pallas import tpu_sc as plsc

HIDDEN = 1024
FF = 2048
E = 8
T = 2048
BT = 256
NA = 2 * T
NB = NA // BT + E
NR = NB * BT
NC, NS = 2, 16
NW = NC * NS
CHUNK = 32


def _route_body(x_ref, rw_ref, xs_ref, pos_ref, be_ref, act_ref):
    x = x_ref[...]
    rw = rw_ref[...]
    logits = lax.dot_general(
        x.astype(jnp.bfloat16), rw.astype(jnp.bfloat16),
        (((1,), (1,)), ((), ())),
        preferred_element_type=jnp.float32)
    col = lax.broadcasted_iota(jnp.int32, (T, E), 1)
    m1 = jnp.max(logits, axis=1, keepdims=True)
    idx1 = jnp.min(jnp.where(logits == m1, col, E), axis=1, keepdims=True)
    sel1 = col == idx1
    neg = jnp.where(sel1, -jnp.inf, logits)
    m2 = jnp.max(neg, axis=1, keepdims=True)
    idx2 = jnp.min(jnp.where(neg == m2, col, E), axis=1, keepdims=True)
    sel2 = col == idx2
    xs_ref[0:T, :] = x * jax.nn.sigmoid(m1)
    xs_ref[T:2 * T, :] = x * jax.nn.sigmoid(m2)

    msel = (sel1 | sel2).astype(jnp.float32)
    C = 256
    carry = jnp.zeros((1, E), jnp.float32)
    ranks = []
    li = lax.broadcasted_iota(jnp.int32, (C, C), 0)
    lj = lax.broadcasted_iota(jnp.int32, (C, C), 1)
    ltri = (lj < li).astype(jnp.float32)
    for c in range(T // C):
        blk = msel[c * C:(c + 1) * C, :]
        r = lax.dot_general(ltri, blk, (((1,), (0,)), ((), ())),
                            preferred_element_type=jnp.float32) + carry
        ranks.append(r)
        carry = carry + jnp.sum(blk, axis=0, keepdims=True)
    rank = jnp.concatenate(ranks, axis=0)
    counts = carry
    padded = jnp.ceil(counts * (1.0 / BT)) * BT
    ei = lax.broadcasted_iota(jnp.int32, (E, E), 0)
    ej = lax.broadcasted_iota(jnp.int32, (E, E), 1)
    etri = (ei < ej).astype(jnp.float32)
    start = lax.dot_general(padded, etri, (((1,), (0,)), ((), ())),
                            preferred_element_type=jnp.float32)
    posf = jnp.broadcast_to(start, (T, E)) + rank
    pos1 = jnp.sum(jnp.where(sel1, posf, 0.0), axis=1)
    pos2 = jnp.sum(jnp.where(sel2, posf, 0.0), axis=1)
    pos_ref[0, :] = pos1.astype(jnp.int32)
    pos_ref[1, :] = pos2.astype(jnp.int32)
    bi = (lax.broadcasted_iota(jnp.int32, (NB, E), 0) * BT).astype(jnp.float32)
    be = jnp.sum((jnp.broadcast_to(start, (NB, E)) <= bi).astype(jnp.int32),
                 axis=1) - 1
    be_ref[0, :] = be
    tot = jnp.sum(padded)
    act_ref[0, :] = (bi[:, 0] < tot).astype(jnp.int32)


_route = pl.pallas_call(
    _route_body,
    out_shape=[
        jax.ShapeDtypeStruct((NA, HIDDEN), jnp.float32),
        jax.ShapeDtypeStruct((2, T), jnp.int32),
        jax.ShapeDtypeStruct((1, NB), jnp.int32),
        jax.ShapeDtypeStruct((1, NB), jnp.int32),
    ],
)


def _scatter_rows_body(xsc_hbm, pos_hbm, out_hbm,
                       xv0, xv1, pv0, pv1, pv2, pv3,
                       sl0, sl1, ss0, ss1):
    wid = lax.axis_index("s") * NC + lax.axis_index("c")
    per_w = NA // NW
    wbase = wid * per_w
    nst = per_w // CHUNK
    xvs = [xv0, xv1]
    pvs = [pv0, pv1, pv2, pv3]
    sls = [sl0, sl1]
    sss = [ss0, ss1]
    for i in range(nst):
        pltpu.sync_copy(pos_hbm.at[pl.ds(wbase + i * CHUNK, CHUNK)], pvs[i])
    ld = [None] * nst
    st = [None] * nst
    ld[0] = pltpu.async_copy(xsc_hbm.at[pl.ds(wbase, CHUNK)], xv0, sl0)
    for sub in range(nst):
        par = sub % 2
        if sub + 1 < nst:
            if sub >= 1:
                st[sub - 1].wait()
            ld[sub + 1] = pltpu.async_copy(
                xsc_hbm.at[pl.ds(wbase + (sub + 1) * CHUNK, CHUNK)],
                xvs[(sub + 1) % 2], sls[(sub + 1) % 2])
        ld[sub].wait()
        st[sub] = pltpu.async_copy(xvs[par], out_hbm.at[pvs[sub]], sss[par])
    st[nst - 2].wait()
    st[nst - 1].wait()


CHC = 8


def _combine_body(y_hbm, p1_hbm, p2_hbm, sh_hbm, out_hbm,
                  sh0, sh1, a0, a1, b0, b1, p1v, p2v,
                  sl0, sl1, ss0, ss1):
    wid = lax.axis_index("s") * NC + lax.axis_index("c")
    per_w = T // NW
    wbase = wid * per_w
    nst = per_w // CHC
    shs = [sh0, sh1]
    avs = [a0, a1]
    bvs = [b0, b1]
    sls = [sl0, sl1]
    sss = [ss0, ss1]
    pltpu.sync_copy(p1_hbm.at[pl.ds(wbase, per_w)], p1v)
    pltpu.sync_copy(p2_hbm.at[pl.ds(wbase, per_w)], p2v)

    def issue(s):
        par = s % 2
        base = wbase + s * CHC
        c1 = pltpu.async_copy(sh_hbm.at[pl.ds(base, CHC)], shs[par], sls[par])
        c2 = pltpu.async_copy(y_hbm.at[p1v.at[pl.ds(s * CHC, CHC)]],
                              avs[par], sls[par])
        c3 = pltpu.async_copy(y_hbm.at[p2v.at[pl.ds(s * CHC, CHC)]],
                              bvs[par], sls[par])
        return (c1, c2, c3)

    ld = [None] * nst
    st = [None] * nst
    ld[0] = issue(0)
    for s in range(nst):
        par = s % 2
        if s + 1 < nst:
            if s >= 1:
                st[s - 1].wait()
            ld[s + 1] = issue(s + 1)
        for c in ld[s]:
            c.wait()
        shp, ap, bp = shs[par], avs[par], bvs[par]

        def row(i, _):
            def vec(d, _):
                sl = pl.ds(d * 16, 16)
                shp[i, sl] = shp[i, sl] + ap[i, sl] + bp[i, sl]
                return 0
            return lax.fori_loop(0, HIDDEN // 16, vec, 0)

        lax.fori_loop(0, CHC, row, 0)
        st[s] = pltpu.async_copy(
            shp, out_hbm.at[pl.ds(wbase + s * CHC, CHC)], sss[par])
    st[nst - 2].wait()
    st[nst - 1].wait()


@functools.cache
def _sc_kernels():
    mesh = plsc.VectorSubcoreMesh(core_axis_name="c", subcore_axis_name="s")
    scatter_rows = pl.kernel(
        _scatter_rows_body, mesh=mesh,
        out_type=jax.ShapeDtypeStruct((NR, HIDDEN), jnp.float32),
        scratch_types=[
            pltpu.VMEM((CHUNK, HIDDEN), jnp.float32),
            pltpu.VMEM((CHUNK, HIDDEN), jnp.float32),
            pltpu.VMEM((CHUNK,), jnp.int32),
            pltpu.VMEM((CHUNK,), jnp.int32),
            pltpu.VMEM((CHUNK,), jnp.int32),
            pltpu.VMEM((CHUNK,), jnp.int32),
            pltpu.SemaphoreType.DMA,
            pltpu.SemaphoreType.DMA,
            pltpu.SemaphoreType.DMA,
            pltpu.SemaphoreType.DMA,
        ],
    )
    combine = pl.kernel(
        _combine_body, mesh=mesh,
        out_type=jax.ShapeDtypeStruct((T, HIDDEN), jnp.float32),
        scratch_types=[
            pltpu.VMEM((CHC, HIDDEN), jnp.float32),
            pltpu.VMEM((CHC, HIDDEN), jnp.float32),
            pltpu.VMEM((CHC, HIDDEN), jnp.float32),
            pltpu.VMEM((CHC, HIDDEN), jnp.float32),
            pltpu.VMEM((CHC, HIDDEN), jnp.float32),
            pltpu.VMEM((CHC, HIDDEN), jnp.float32),
            pltpu.VMEM((T // NW,), jnp.int32),
            pltpu.VMEM((T // NW,), jnp.int32),
            pltpu.SemaphoreType.DMA,
            pltpu.SemaphoreType.DMA,
            pltpu.SemaphoreType.DMA,
            pltpu.SemaphoreType.DMA,
        ],
    )
    return scatter_rows, combine


def _experts_body(be_ref, act_ref, xs_ref, gu_ref, dn_ref, y_ref):
    b = pl.program_id(0)

    @pl.when(act_ref[b] != 0)
    def _():
        xb = xs_ref[...]
        gu = lax.dot_general(xb, gu_ref[0], (((1,), (0,)), ((), ())),
                             preferred_element_type=jnp.float32)
        gate = gu[:, :FF]
        up = gu[:, FF:]
        h = up * (gate * jax.nn.sigmoid(gate))
        y_ref[...] = lax.dot_general(h, dn_ref[0], (((1,), (0,)), ((), ())),
                                     preferred_element_type=jnp.float32)


_experts = pl.pallas_call(
    _experts_body,
    grid_spec=pltpu.PrefetchScalarGridSpec(
        num_scalar_prefetch=2,
        grid=(NB,),
        in_specs=[
            pl.BlockSpec((BT, HIDDEN), lambda b, be, act: (b, 0)),
            pl.BlockSpec((1, HIDDEN, 2 * FF), lambda b, be, act: (be[b], 0, 0)),
            pl.BlockSpec((1, FF, HIDDEN), lambda b, be, act: (be[b], 0, 0)),
        ],
        out_specs=pl.BlockSpec((BT, HIDDEN), lambda b, be, act: (b, 0)),
    ),
    out_shape=jax.ShapeDtypeStruct((NR, HIDDEN), jnp.float32),
)


def _shared_body(x_ref, g_ref, u_ref, d_ref, o_ref):
    xb = x_ref[...]
    g = lax.dot_general(xb, g_ref[...], (((1,), (1,)), ((), ())),
                        preferred_element_type=jnp.float32)
    u = lax.dot_general(xb, u_ref[...], (((1,), (1,)), ((), ())),
                        preferred_element_type=jnp.float32)
    h = (g * jax.nn.sigmoid(g)) * u
    o_ref[...] = lax.dot_general(h, d_ref[...], (((1,), (1,)), ((), ())),
                                 preferred_element_type=jnp.float32)


_BTS = 512
_shared = pl.pallas_call(
    _shared_body,
    grid=(T // _BTS,),
    in_specs=[
        pl.BlockSpec((_BTS, HIDDEN), lambda i: (i, 0)),
        pl.BlockSpec((FF, HIDDEN), lambda i: (0, 0)),
        pl.BlockSpec((FF, HIDDEN), lambda i: (0, 0)),
        pl.BlockSpec((HIDDEN, FF), lambda i: (0, 0)),
    ],
    out_specs=pl.BlockSpec((_BTS, HIDDEN), lambda i: (i, 0)),
    out_shape=jax.ShapeDtypeStruct((T, HIDDEN), jnp.float32),
)


def kernel(hidden_states, router_w, gate_up_proj, down_proj,
           shared_gate_w, shared_up_w, shared_down_w):
    x = hidden_states.reshape(T, HIDDEN)
    scatter_rows, combine = _sc_kernels()
    xsc, posarr, be, act = _route(x, router_w)
    xs = scatter_rows(xsc, posarr.reshape(NA))
    sh = _shared(x, shared_gate_w, shared_up_w, shared_down_w)
    y = _experts(be.reshape(NB), act.reshape(NB), xs, gate_up_proj, down_proj)
    out = combine(y, posarr[0], posarr[1], sh)
    return out.reshape(1, T, HIDDEN)
